# baseline (speedup 1.0000x reference)
import jax
import jax.numpy as jnp
from jax.experimental import pallas as pl

_B, _C, _S = 256, 768, 196
_K = _C * _S
_NC = 10
_KB = 6272
_NSTEP = _K // _KB


def _body(f_ref, w_ref, b_ref, o_ref):
    i = pl.program_id(0)

    @pl.when(i == 0)
    def _init():
        o_ref[...] = jnp.broadcast_to(b_ref[...], o_ref.shape)

    o_ref[...] += jax.lax.dot_general(
        f_ref[...], w_ref[...], (((1,), (0,)), ((), ())),
        preferred_element_type=jnp.float32)


def kernel(features, W, b):
    f2 = features.reshape(_B, _K)
    w_exp = (jnp.repeat(W, _S, axis=1) * (1.0 / _S)).T
    out = pl.pallas_call(
        _body,
        grid=(_NSTEP,),
        in_specs=[
            pl.BlockSpec((_B, _KB), lambda i: (0, i)),
            pl.BlockSpec((_KB, _NC), lambda i: (i, 0)),
            pl.BlockSpec((1, _NC), lambda i: (0, 0)),
        ],
        out_specs=pl.BlockSpec((_B, _NC), lambda i: (0, 0)),
        out_shape=jax.ShapeDtypeStruct((_B, _NC), jnp.float32),
    )(f2, w_exp, b.reshape(1, _NC))
    return out


# speedup vs baseline: 8.4158x; 8.4158x over previous
import jax
import jax.numpy as jnp
from jax.experimental import pallas as pl
from jax.experimental.pallas import tpu as pltpu

_B, _C, _S = 256, 768, 196
_NC = 10
_SB = 4
_NSTEP = _S // _SB


def _body(f_ref, w_ref, b_ref, o_ref, acc_ref):
    i = pl.program_id(0)
    partial = jnp.sum(f_ref[...], axis=0)

    @pl.when(i == 0)
    def _init():
        acc_ref[...] = partial

    @pl.when(i > 0)
    def _acc():
        acc_ref[...] += partial

    @pl.when(i == _NSTEP - 1)
    def _fin():
        pooled = acc_ref[...] * (1.0 / _S)
        o_ref[...] = jax.lax.dot_general(
            pooled, w_ref[...], (((1,), (1,)), ((), ())),
            preferred_element_type=jnp.float32) + b_ref[...]


def kernel(features, W, b):
    f3 = features.transpose(2, 3, 0, 1).reshape(_S, _B, _C)
    out = pl.pallas_call(
        _body,
        grid=(_NSTEP,),
        in_specs=[
            pl.BlockSpec((_SB, _B, _C), lambda i: (i, 0, 0)),
            pl.BlockSpec((_NC, _C), lambda i: (0, 0)),
            pl.BlockSpec((1, _NC), lambda i: (0, 0)),
        ],
        out_specs=pl.BlockSpec((_B, _NC), lambda i: (0, 0)),
        out_shape=jax.ShapeDtypeStruct((_B, _NC), jnp.float32),
        scratch_shapes=[pltpu.VMEM((_B, _C), jnp.float32)],
    )(f3, W, b.reshape(1, _NC))
    return out


# speedup vs baseline: 10.6500x; 1.2655x over previous
import jax
import jax.numpy as jnp
from jax.experimental import pallas as pl
from jax.experimental.pallas import tpu as pltpu

_B, _C, _S = 256, 768, 196
_NC = 10
_SB = 14
_NSTEP = _S // _SB


def _body(f_ref, w_ref, b_ref, o_ref, acc_ref):
    i = pl.program_id(0)
    partial = jnp.sum(f_ref[...], axis=0)

    @pl.when(i == 0)
    def _init():
        acc_ref[...] = partial

    @pl.when(i > 0)
    def _acc():
        acc_ref[...] += partial

    @pl.when(i == _NSTEP - 1)
    def _fin():
        pooled = acc_ref[...] * (1.0 / _S)
        o_ref[...] = jax.lax.dot_general(
            pooled, w_ref[...], (((1,), (1,)), ((), ())),
            preferred_element_type=jnp.float32) + b_ref[...]


def kernel(features, W, b):
    f3 = features.transpose(2, 3, 0, 1).reshape(_S, _B, _C)
    out = pl.pallas_call(
        _body,
        grid=(_NSTEP,),
        in_specs=[
            pl.BlockSpec((_SB, _B, _C), lambda i: (i, 0, 0)),
            pl.BlockSpec((_NC, _C), lambda i: (0, 0)),
            pl.BlockSpec((1, _NC), lambda i: (0, 0)),
        ],
        out_specs=pl.BlockSpec((_B, _NC), lambda i: (0, 0)),
        out_shape=jax.ShapeDtypeStruct((_B, _NC), jnp.float32),
        scratch_shapes=[pltpu.VMEM((_B, _C), jnp.float32)],
    )(f3, W, b.reshape(1, _NC))
    return out
